# Initial kernel scaffold; baseline (speedup 1.0000x reference)
import jax
import jax.numpy as jnp
from jax.experimental import pallas as pl


def kernel(x, edge_index, W1, b1, g1, be1, W2, b2, g2, be2, W3, b3):
    raise NotImplementedError("write your pallas kernel here")


# speedup vs baseline: 6.6587x; 6.6587x over previous
import functools
import math

import jax
import jax.numpy as jnp
from jax import lax
from jax.experimental import pallas as pl
from jax.experimental.pallas import tpu as pltpu
from jax.experimental.pallas import tpu_sc as plsc

N = 10000
E = 320000
D_IN = 128
D_H = 128
D_OUT = 40
D3P = 64

NPAD = 10240
EPAD = 327680
ZROW = N + 128
NC, NS = 2, 16
NW = NC * NS
BNC = 1.0 / math.sqrt(1.0 + 1e-5)

BM = 256
_GRID = NPAD // BM


def _make_sc_agg(D):
    EPT = EPAD // NW
    CH = 256
    NCHUNK = EPT // CH
    RPT = NPAD // NS
    mesh = plsc.VectorSubcoreMesh(core_axis_name="c", subcore_axis_name="s")

    @functools.partial(
        pl.kernel,
        mesh=mesh,
        out_type=(jax.ShapeDtypeStruct((NPAD, D), jnp.float32),
                  jax.ShapeDtypeStruct((NPAD, D), jnp.float32)),
        scratch_types=[
            pltpu.VMEM((CH,), jnp.int32),
            pltpu.VMEM((CH,), jnp.int32),
            pltpu.VMEM((CH, D), jnp.float32),
            pltpu.VMEM_SHARED((NPAD, D), jnp.float32),
            pltpu.SemaphoreType.DMA,
        ],
    )
    def agg_kernel(hs_hbm, zeros_hbm, src_hbm, dst_hbm, out_a, out_b,
                   sidx, didx, rows, acc, sem):
        c = lax.axis_index("c")
        s = lax.axis_index("s")
        rbase = s * RPT

        @pl.when(c == 0)
        def _():
            pltpu.sync_copy(hs_hbm.at[pl.ds(rbase, RPT)],
                            acc.at[pl.ds(rbase, RPT)])

        @pl.when(c != 0)
        def _():
            pltpu.sync_copy(zeros_hbm.at[pl.ds(rbase, RPT)],
                            acc.at[pl.ds(rbase, RPT)])

        plsc.subcore_barrier()

        ebase = (c * NS + s) * EPT

        def body(i, carry):
            b = ebase + i * CH
            pltpu.sync_copy(src_hbm.at[pl.ds(b, CH)], sidx)
            pltpu.sync_copy(dst_hbm.at[pl.ds(b, CH)], didx)
            pltpu.async_copy(hs_hbm.at[sidx], rows, sem).wait()
            pltpu.sync_copy(rows, acc.at[didx], add=True)
            return carry

        lax.fori_loop(0, NCHUNK, body, 0)
        plsc.subcore_barrier()

        @pl.when(c == 0)
        def _():
            pltpu.sync_copy(acc.at[pl.ds(rbase, RPT)],
                            out_a.at[pl.ds(rbase, RPT)])

        @pl.when(c != 0)
        def _():
            pltpu.sync_copy(acc.at[pl.ds(rbase, RPT)],
                            out_b.at[pl.ds(rbase, RPT)])

    return agg_kernel


def _make_sc_deg():
    EPT = EPAD // NW
    CH = 1024
    NCHUNK = EPT // CH
    RPT = NPAD // NS
    mesh = plsc.VectorSubcoreMesh(core_axis_name="c", subcore_axis_name="s")

    @functools.partial(
        pl.kernel,
        mesh=mesh,
        out_type=(jax.ShapeDtypeStruct((NPAD,), jnp.float32),
                  jax.ShapeDtypeStruct((NPAD,), jnp.float32)),
        scratch_types=[
            pltpu.VMEM((CH,), jnp.int32),
            pltpu.VMEM((CH,), jnp.float32),
            pltpu.VMEM_SHARED((NPAD,), jnp.float32),
        ],
    )
    def deg_kernel(dst_hbm, zeros_hbm, out_a, out_b, didx, ones, hist):
        c = lax.axis_index("c")
        s = lax.axis_index("s")
        rbase = s * RPT

        def fill(i, carry):
            ones[pl.ds(i * 16, 16)] = jnp.ones((16,), jnp.float32)
            return carry

        lax.fori_loop(0, CH // 16, fill, 0)
        pltpu.sync_copy(zeros_hbm.at[pl.ds(rbase, RPT)],
                        hist.at[pl.ds(rbase, RPT)])
        plsc.subcore_barrier()

        ebase = (c * NS + s) * EPT

        def body(i, carry):
            pltpu.sync_copy(dst_hbm.at[pl.ds(ebase + i * CH, CH)], didx)
            pltpu.sync_copy(ones, hist.at[didx], add=True)
            return carry

        lax.fori_loop(0, NCHUNK, body, 0)
        plsc.subcore_barrier()

        @pl.when(c == 0)
        def _():
            pltpu.sync_copy(hist.at[pl.ds(rbase, RPT)],
                            out_a.at[pl.ds(rbase, RPT)])

        @pl.when(c != 0)
        def _():
            pltpu.sync_copy(hist.at[pl.ds(rbase, RPT)],
                            out_b.at[pl.ds(rbase, RPT)])

    return deg_kernel


def _row_spec(d):
    return pl.BlockSpec((BM, d), lambda i: (i, 0))


def _full_spec(r, d):
    return pl.BlockSpec((r, d), lambda i: (0, 0))


def _tc1_body(x_ref, w_ref, d0_ref, d1_ref, hs_ref, dinv_ref):
    deg = d0_ref[...] + d1_ref[...] + 1.0
    dv = lax.rsqrt(deg)
    dinv_ref[...] = dv
    h = jnp.dot(x_ref[...], w_ref[...], preferred_element_type=jnp.float32)
    hs_ref[...] = h * dv


def _tc1(x_pad, w1, deg0, deg1):
    return pl.pallas_call(
        _tc1_body,
        grid=(_GRID,),
        in_specs=[_row_spec(D_IN), _full_spec(D_IN, D_H),
                  _row_spec(1), _row_spec(1)],
        out_specs=[_row_spec(D_H), _row_spec(1)],
        out_shape=[jax.ShapeDtypeStruct((NPAD, D_H), jnp.float32),
                   jax.ShapeDtypeStruct((NPAD, 1), jnp.float32)],
    )(x_pad, w1, deg0, deg1)


def _epilogue(aggA, aggB, dv, b, g, be, i):
    z = (aggA + aggB) * dv + b
    u = jnp.maximum(z, 0.0) * (g * BNC) + be
    rows = i * BM + lax.broadcasted_iota(jnp.int32, (BM, 1), 0)
    return jnp.where(rows < N, u, 0.0) * dv


def _tc_mid_body(aggA_ref, aggB_ref, dinv_ref, b_ref, g_ref, be_ref, w_ref,
                 hs_ref):
    u = _epilogue(aggA_ref[...], aggB_ref[...], dinv_ref[...], b_ref[...],
                  g_ref[...], be_ref[...], pl.program_id(0))
    hs_ref[...] = jnp.dot(u, w_ref[...], preferred_element_type=jnp.float32)


def _tc_mid(aggA, aggB, dinv, b, g, be, w_next):
    return pl.pallas_call(
        _tc_mid_body,
        grid=(_GRID,),
        in_specs=[_row_spec(D_H), _row_spec(D_H), _row_spec(1),
                  _full_spec(1, D_H), _full_spec(1, D_H), _full_spec(1, D_H),
                  _full_spec(D_H, D_H)],
        out_specs=[_row_spec(D_H)],
        out_shape=[jax.ShapeDtypeStruct((NPAD, D_H), jnp.float32)],
    )(aggA, aggB, dinv, b, g, be, w_next)[0]


def _tc_pre3_body(aggA_ref, aggB_ref, dinv_ref, b_ref, g_ref, be_ref, q_ref):
    q_ref[...] = _epilogue(aggA_ref[...], aggB_ref[...], dinv_ref[...],
                           b_ref[...], g_ref[...], be_ref[...],
                           pl.program_id(0))


def _tc_pre3(aggA, aggB, dinv, b, g, be):
    return pl.pallas_call(
        _tc_pre3_body,
        grid=(_GRID,),
        in_specs=[_row_spec(D_H), _row_spec(D_H), _row_spec(1),
                  _full_spec(1, D_H), _full_spec(1, D_H), _full_spec(1, D_H)],
        out_specs=[_row_spec(D_H)],
        out_shape=[jax.ShapeDtypeStruct((NPAD, D_H), jnp.float32)],
    )(aggA, aggB, dinv, b, g, be)[0]


def _tc_fin_body(aggA_ref, aggB_ref, dinv_ref, w_ref, b_ref, out_ref):
    h = (aggA_ref[...] + aggB_ref[...]) * dinv_ref[...]
    out_ref[...] = jnp.dot(h, w_ref[...],
                           preferred_element_type=jnp.float32) + b_ref[...]


def _tc_fin(aggA, aggB, dinv, w3p, b3p):
    return pl.pallas_call(
        _tc_fin_body,
        grid=(_GRID,),
        in_specs=[_row_spec(D_H), _row_spec(D_H), _row_spec(1),
                  _full_spec(D_H, D3P), _full_spec(1, D3P)],
        out_specs=[_row_spec(D3P)],
        out_shape=[jax.ShapeDtypeStruct((NPAD, D3P), jnp.float32)],
    )(aggA, aggB, dinv, w3p, b3p)[0]


_sc_agg128 = _make_sc_agg(D_H)
_sc_deg = _make_sc_deg()


def kernel(x, edge_index, W1, b1, g1, be1, W2, b2, g2, be2, W3, b3):
    f32 = jnp.float32
    x_pad = jnp.pad(x, ((0, NPAD - N), (0, 0)))
    epad = jnp.full((EPAD - E,), ZROW, jnp.int32)
    srcp = jnp.concatenate([edge_index[0], epad])
    dstp = jnp.concatenate([edge_index[1], epad])

    zeros128 = jnp.zeros((NPAD, D_H), f32)
    zeros1 = jnp.zeros((NPAD,), f32)
    w3p = jnp.pad(W3, ((0, 0), (0, D3P - D_OUT)))
    b3p = jnp.pad(b3, (0, D3P - D_OUT)).reshape(1, D3P)

    deg0, deg1 = _sc_deg(dstp, zeros1)
    deg0 = deg0.reshape(NPAD, 1)
    deg1 = deg1.reshape(NPAD, 1)

    hs1, dinv = _tc1(x_pad, W1, deg0, deg1)

    a1, p1 = _sc_agg128(hs1, zeros128, srcp, dstp)
    hs2 = _tc_mid(a1, p1, dinv, b1.reshape(1, D_H), g1.reshape(1, D_H),
                  be1.reshape(1, D_H), W2)

    a2, p2 = _sc_agg128(hs2, zeros128, srcp, dstp)
    q3 = _tc_pre3(a2, p2, dinv, b2.reshape(1, D_H), g2.reshape(1, D_H),
                  be2.reshape(1, D_H))

    a3, p3 = _sc_agg128(q3, zeros128, srcp, dstp)
    out64 = _tc_fin(a3, p3, dinv, w3p, b3p)
    return out64[:N, :D_OUT]
